# baseline (speedup 1.0000x reference)
import functools

import jax
import jax.numpy as jnp
from jax import lax
from jax.experimental import pallas as pl
from jax.experimental.pallas import tpu as pltpu
from jax.experimental.pallas import tpu_sc as plsc

_B, _L, _D = 4, 2048, 1024
_E, _K, _KWIN = 16, 2, 25
_PAD = (_KWIN - 1) // 2
_NT = _B * _L
_NSUB = 16
_TPT = _NT // _NSUB
_CAP = 8.0
_ZR = 128


_CH = 512


def _ln(h, g, b):
    mu = jnp.mean(h, axis=1, keepdims=True)
    d = h - mu
    var = jnp.mean(d * d, axis=1, keepdims=True)
    return d * lax.rsqrt(var + 1e-5) * g + b


def _fused_body(x_ref, Wt_ref, bt_ref, gt_ref, bet_ref, Ws_ref, bs_ref,
                gs_ref, bes_ref, W2_ref, b2_ref, med_ref):
    xb = x_ref[0]
    bt = bt_ref[...]
    gt = gt_ref[...]
    bet = bet_ref[...]
    bs = bs_ref[...]
    gs = gs_ref[...]
    bes = bes_ref[...]
    W2 = W2_ref[...]
    b2 = b2_ref[...]
    nch = _L // _CH
    for c in range(nch):
        lo, hi = c * _CH - _PAD, (c + 1) * _CH + _PAD
        seg = xb[max(lo, 0):min(hi, _L)]
        h1 = jnp.dot(seg, Wt_ref[...], preferred_element_type=jnp.float32)
        h2 = jnp.dot(seg, Ws_ref[...], preferred_element_type=jnp.float32)
        if c == 0:
            h1 = jnp.concatenate([jnp.broadcast_to(h1[0:1], (_PAD, _D)), h1], axis=0)
            h2 = jnp.concatenate([jnp.broadcast_to(h2[0:1], (_PAD, _D)), h2], axis=0)
        if c == nch - 1:
            h1 = jnp.concatenate([h1, jnp.broadcast_to(h1[-1:], (_PAD, _D))], axis=0)
            h2 = jnp.concatenate([h2, jnp.broadcast_to(h2[-1:], (_PAD, _D))], axis=0)
        n5 = _CH + 4 * 5
        s5a = h1[0:n5] + h1[1:n5 + 1] + h1[2:n5 + 2] + h1[3:n5 + 3] + h1[4:n5 + 4]
        m1 = (s5a[0:_CH] + s5a[5:_CH + 5] + s5a[10:_CH + 10]
              + s5a[15:_CH + 15] + s5a[20:_CH + 20]) * (1.0 / _KWIN)
        s5b = h2[0:n5] + h2[1:n5 + 1] + h2[2:n5 + 2] + h2[3:n5 + 3] + h2[4:n5 + 4]
        m2 = (s5b[0:_CH] + s5b[5:_CH + 5] + s5b[10:_CH + 10]
              + s5b[15:_CH + 15] + s5b[20:_CH + 20]) * (1.0 / _KWIN)
        ti = _ln(h1[_PAD:_PAD + _CH] - m1 + bt, gt, bet)
        si = _ln(m2 + bs, gs, bes)
        a = jnp.maximum(ti + si, 0.0)
        med_ref[pl.ds(c * _CH, _CH), :] = (
            jnp.dot(a, W2, preferred_element_type=jnp.float32) + b2)


def _fused_dense(x, Wt, bt, gt, bet, Ws, bs, gs, bes, W2, b2, interpret=False):
    full = lambda b: (0, 0)
    return pl.pallas_call(
        _fused_body,
        grid=(_B,),
        in_specs=[
            pl.BlockSpec((1, _L, _D), lambda b: (b, 0, 0)),
            pl.BlockSpec((_D, _D), full),
            pl.BlockSpec((1, _D), full),
            pl.BlockSpec((1, _D), full),
            pl.BlockSpec((1, _D), full),
            pl.BlockSpec((_D, _D), full),
            pl.BlockSpec((1, _D), full),
            pl.BlockSpec((1, _D), full),
            pl.BlockSpec((1, _D), full),
            pl.BlockSpec((_D, _E), full),
            pl.BlockSpec((1, _E), full),
        ],
        out_specs=pl.BlockSpec((_L, _E), lambda b: (b, 0)),
        out_shape=jax.ShapeDtypeStruct((_NT, _E), jnp.float32),
        interpret=interpret,
    )(x, Wt, bt, gt, bet, Ws, bs, gs, bes, W2, b2)


def _mavg_body(x_ref, o_ref):
    xb = x_ref[0]
    f = xb.shape[1]
    xp = jnp.concatenate(
        [jnp.broadcast_to(xb[0:1], (_PAD, f)), xb,
         jnp.broadcast_to(xb[_L - 1:_L], (_PAD, f))], axis=0)
    acc = xp[0:_L]
    for d in range(1, _KWIN):
        acc = acc + xp[d:d + _L]
    o_ref[0] = acc * (1.0 / _KWIN)


def _moving_mean(x, interpret=False):
    f = 512
    return pl.pallas_call(
        _mavg_body,
        grid=(_B, _D // f),
        in_specs=[pl.BlockSpec((1, _L, f), lambda b, i: (b, 0, i))],
        out_specs=pl.BlockSpec((1, _L, f), lambda b, i: (b, 0, i)),
        out_shape=jax.ShapeDtypeStruct((_B, _L, _D), jnp.float32),
        interpret=interpret,
    )(x)


_T = 512


def _dense_body(x_ref, mm_ref, Wt_ref, bt_ref, gt_ref, bet_ref,
                Ws_ref, bs_ref, gs_ref, bes_ref, W2_ref, b2_ref, med_ref):
    mmc = mm_ref[...]
    res = x_ref[...] - mmc
    h1 = jnp.dot(res, Wt_ref[...], preferred_element_type=jnp.float32) + bt_ref[...]
    mu1 = jnp.mean(h1, axis=1, keepdims=True)
    d1 = h1 - mu1
    v1 = jnp.mean(d1 * d1, axis=1, keepdims=True)
    ti = d1 * lax.rsqrt(v1 + 1e-5) * gt_ref[...] + bet_ref[...]
    h2 = jnp.dot(mmc, Ws_ref[...], preferred_element_type=jnp.float32) + bs_ref[...]
    mu2 = jnp.mean(h2, axis=1, keepdims=True)
    d2 = h2 - mu2
    v2 = jnp.mean(d2 * d2, axis=1, keepdims=True)
    si = d2 * lax.rsqrt(v2 + 1e-5) * gs_ref[...] + bes_ref[...]
    a = jnp.maximum(ti + si, 0.0)
    med_ref[...] = jnp.dot(a, W2_ref[...], preferred_element_type=jnp.float32) + b2_ref[...]


def _dense(x2, mm2, Wt, bt, gt, bet, Ws, bs, gs, bes, W2, b2, interpret=False):
    full = lambda i: (0, 0)
    return pl.pallas_call(
        _dense_body,
        grid=(_NT // _T,),
        in_specs=[
            pl.BlockSpec((_T, _D), lambda i: (i, 0)),
            pl.BlockSpec((_T, _D), lambda i: (i, 0)),
            pl.BlockSpec((_D, _D), full),
            pl.BlockSpec((1, _D), full),
            pl.BlockSpec((1, _D), full),
            pl.BlockSpec((1, _D), full),
            pl.BlockSpec((_D, _D), full),
            pl.BlockSpec((1, _D), full),
            pl.BlockSpec((1, _D), full),
            pl.BlockSpec((1, _D), full),
            pl.BlockSpec((_D, _E), full),
            pl.BlockSpec((1, _E), full),
        ],
        out_specs=pl.BlockSpec((_T, _E), lambda i: (i, 0)),
        out_shape=jax.ShapeDtypeStruct((_NT, _E), jnp.float32),
        interpret=interpret,
    )(x2, mm2, Wt, bt, gt, bet, Ws, bs, gs, bes, W2, b2)


@functools.cache
def _build_route_kernel():
    sc_mesh = plsc.VectorSubcoreMesh(
        core_axis_name="c", subcore_axis_name="s",
        num_cores=1, num_subcores=_NSUB)
    return functools.partial(
        pl.kernel,
        out_type=jax.ShapeDtypeStruct((_NT, _E), jnp.float32),
        mesh=sc_mesh,
        compiler_params=pltpu.CompilerParams(needs_layout_passes=False),
        scratch_types=[
        pltpu.VMEM((_TPT, _E), jnp.float32),
        pltpu.VMEM((2, _E), jnp.float32),
        pltpu.VMEM((_NSUB, 2, _E), jnp.float32),
        pltpu.VMEM((_E, _E), jnp.float32),
        pltpu.VMEM((_E, _E), jnp.float32),
        pltpu.VMEM((_E, _E), jnp.float32),
        pltpu.VMEM((_ZR, _E), jnp.float32),
        pltpu.VMEM_SHARED((_NSUB, 2, _E), jnp.float32),
        ],
    )(_route_body)


def _route_body(med_hbm, out_hbm, med_v, selp_v, comb_v, rows_v, g_v,
                ob_v, z_v, shared):
    wid = lax.axis_index("s")
    base = wid * _TPT
    pltpu.sync_copy(med_hbm.at[pl.ds(base, _TPT)], med_v)

    iota = lax.iota(jnp.int32, _E)
    zero = jnp.zeros((_E,), jnp.float32)
    neg = jnp.full((_E,), -3.4e38, jnp.float32)

    def tok(i, carry):
        s1, s2 = carry
        v = med_v[i]
        m1 = jnp.max(v)
        i1 = plsc.all_reduce_ffs(v == m1)
        oh1 = iota == i1
        v2 = jnp.where(oh1, neg, v)
        m2 = jnp.max(v2)
        i2 = plsc.all_reduce_ffs(v2 == m2)
        oh2 = iota == i2
        return jnp.where(oh1, 1.0, s1), jnp.where(oh2, 1.0, s2)

    s1, s2 = lax.fori_loop(0, _TPT, tok, (zero, zero), unroll=4)
    selp_v[0] = s1
    selp_v[1] = s2
    pltpu.sync_copy(selp_v, shared.at[wid])

    def zb(i, _):
        z_v[i] = zero
        return 0
    lax.fori_loop(0, _ZR, zb, 0, unroll=8)
    for k in range(_TPT // _ZR):
        pltpu.sync_copy(z_v, out_hbm.at[pl.ds(base + k * _ZR, _ZR)])

    plsc.subcore_barrier()

    @pl.when(wid == 0)
    def _finish():
        pltpu.sync_copy(shared, comb_v)
        tpb = _NSUB // _B
        den = [jnp.full((_E,), 1e-4, jnp.float32),
               jnp.full((_E,), 1e-4, jnp.float32)]
        ts = []
        for b in range(_B):
            sel = []
            for j in range(_K):
                acc = comb_v[b * tpb, j]
                for t in range(1, tpb):
                    acc = jnp.maximum(acc, comb_v[b * tpb + t, j])
                sel.append(acc)
            pltpu.sync_copy(med_hbm.at[pl.ds(b * _L, _E)], rows_v)
            for l in range(_E):
                v = rows_v[l]
                e = jnp.exp(v - jnp.max(v))
                g_v[l] = e / jnp.sum(e)
            tb = []
            for j in range(_K):
                colj = plsc.load_gather(
                    g_v, [iota, jnp.full((_E,), j, jnp.int32)])
                tj = colj * sel[j]
                den[j] = den[j] + tj
                tb.append(tj)
            ts.append(tb)
        for b in range(_B):
            for l in range(_E):
                ob_v[l] = zero
            for j in range(_K):
                oj = ts[b][j] / den[j] * _CAP
                plsc.store_scatter(
                    ob_v, [iota, jnp.full((_E,), j, jnp.int32)], oj)
            pltpu.sync_copy(ob_v, out_hbm.at[pl.ds(b * _L, _E)])


def kernel(x, Wt, bt, gt, bet, Ws, bs, gs, bes, W2, b2):
    med = _fused_dense(x, Wt, bt.reshape(1, _D), gt.reshape(1, _D),
                       bet.reshape(1, _D), Ws, bs.reshape(1, _D),
                       gs.reshape(1, _D), bes.reshape(1, _D), W2,
                       b2.reshape(1, _E))
    out2 = _build_route_kernel()(med)
    return out2.reshape(_B, _L, _E)


# speedup vs baseline: 1.5135x; 1.5135x over previous
import functools

import jax
import jax.numpy as jnp
from jax import lax
from jax.experimental import pallas as pl
from jax.experimental.pallas import tpu as pltpu
from jax.experimental.pallas import tpu_sc as plsc

_B, _L, _D = 4, 2048, 1024
_E, _K, _KWIN = 16, 2, 25
_PAD = (_KWIN - 1) // 2
_NT = _B * _L
_NSUB = 16
_TPT = _NT // _NSUB
_CAP = 8.0
_ZR = 128


_CH = 512


def _ln(h, g, b):
    mu = jnp.mean(h, axis=1, keepdims=True)
    d = h - mu
    var = jnp.mean(d * d, axis=1, keepdims=True)
    return d * lax.rsqrt(var + 1e-5) * g + b


_MB = 128
_MBH = _MB + _KWIN - 1


def _fused_body(x_ref, Wt_ref, bt_ref, gt_ref, bet_ref, Ws_ref, bs_ref,
                gs_ref, bes_ref, W2_ref, b2_ref, A_ref, med_ref):
    xb = x_ref[0]
    bt = bt_ref[...]
    gt = gt_ref[...]
    bet = bet_ref[...]
    bs = bs_ref[...]
    gs = gs_ref[...]
    bes = bes_ref[...]
    W2 = W2_ref[...]
    b2 = b2_ref[...]
    A = A_ref[...]
    nch = _L // _CH
    for c in range(nch):
        lo, hi = c * _CH - _PAD, (c + 1) * _CH + _PAD
        seg = xb[max(lo, 0):min(hi, _L)]
        h1 = jnp.dot(seg, Wt_ref[...], preferred_element_type=jnp.float32)
        h2 = jnp.dot(seg, Ws_ref[...], preferred_element_type=jnp.float32)
        if c == 0:
            h1 = jnp.concatenate([jnp.broadcast_to(h1[0:1], (_PAD, _D)), h1], axis=0)
            h2 = jnp.concatenate([jnp.broadcast_to(h2[0:1], (_PAD, _D)), h2], axis=0)
        if c == nch - 1:
            h1 = jnp.concatenate([h1, jnp.broadcast_to(h1[-1:], (_PAD, _D))], axis=0)
            h2 = jnp.concatenate([h2, jnp.broadcast_to(h2[-1:], (_PAD, _D))], axis=0)
        m1 = jnp.concatenate(
            [jnp.dot(A, h1[s * _MB:s * _MB + _MBH],
                     preferred_element_type=jnp.float32)
             for s in range(_CH // _MB)], axis=0)
        m2 = jnp.concatenate(
            [jnp.dot(A, h2[s * _MB:s * _MB + _MBH],
                     preferred_element_type=jnp.float32)
             for s in range(_CH // _MB)], axis=0)
        ti = _ln(h1[_PAD:_PAD + _CH] - m1 + bt, gt, bet)
        si = _ln(m2 + bs, gs, bes)
        a = jnp.maximum(ti + si, 0.0)
        med_ref[pl.ds(c * _CH, _CH), :] = (
            jnp.dot(a, W2, preferred_element_type=jnp.float32) + b2)


def _band_matrix():
    i = jnp.arange(_MB)[:, None]
    j = jnp.arange(_MBH)[None, :]
    d = j - i
    return jnp.where((d >= 0) & (d < _KWIN), 1.0 / _KWIN, 0.0).astype(jnp.float32)


def _fused_dense(x, Wt, bt, gt, bet, Ws, bs, gs, bes, W2, b2, interpret=False):
    full = lambda b: (0, 0)
    return pl.pallas_call(
        _fused_body,
        grid=(_B,),
        in_specs=[
            pl.BlockSpec((1, _L, _D), lambda b: (b, 0, 0)),
            pl.BlockSpec((_D, _D), full),
            pl.BlockSpec((1, _D), full),
            pl.BlockSpec((1, _D), full),
            pl.BlockSpec((1, _D), full),
            pl.BlockSpec((_D, _D), full),
            pl.BlockSpec((1, _D), full),
            pl.BlockSpec((1, _D), full),
            pl.BlockSpec((1, _D), full),
            pl.BlockSpec((_D, _E), full),
            pl.BlockSpec((1, _E), full),
            pl.BlockSpec((_MB, _MBH), full),
        ],
        out_specs=pl.BlockSpec((_L, _E), lambda b: (b, 0)),
        out_shape=jax.ShapeDtypeStruct((_NT, _E), jnp.float32),
        interpret=interpret,
    )(x, Wt, bt, gt, bet, Ws, bs, gs, bes, W2, b2, _band_matrix())


def _mavg_body(x_ref, o_ref):
    xb = x_ref[0]
    f = xb.shape[1]
    xp = jnp.concatenate(
        [jnp.broadcast_to(xb[0:1], (_PAD, f)), xb,
         jnp.broadcast_to(xb[_L - 1:_L], (_PAD, f))], axis=0)
    acc = xp[0:_L]
    for d in range(1, _KWIN):
        acc = acc + xp[d:d + _L]
    o_ref[0] = acc * (1.0 / _KWIN)


def _moving_mean(x, interpret=False):
    f = 512
    return pl.pallas_call(
        _mavg_body,
        grid=(_B, _D // f),
        in_specs=[pl.BlockSpec((1, _L, f), lambda b, i: (b, 0, i))],
        out_specs=pl.BlockSpec((1, _L, f), lambda b, i: (b, 0, i)),
        out_shape=jax.ShapeDtypeStruct((_B, _L, _D), jnp.float32),
        interpret=interpret,
    )(x)


_T = 512


def _dense_body(x_ref, mm_ref, Wt_ref, bt_ref, gt_ref, bet_ref,
                Ws_ref, bs_ref, gs_ref, bes_ref, W2_ref, b2_ref, med_ref):
    mmc = mm_ref[...]
    res = x_ref[...] - mmc
    h1 = jnp.dot(res, Wt_ref[...], preferred_element_type=jnp.float32) + bt_ref[...]
    mu1 = jnp.mean(h1, axis=1, keepdims=True)
    d1 = h1 - mu1
    v1 = jnp.mean(d1 * d1, axis=1, keepdims=True)
    ti = d1 * lax.rsqrt(v1 + 1e-5) * gt_ref[...] + bet_ref[...]
    h2 = jnp.dot(mmc, Ws_ref[...], preferred_element_type=jnp.float32) + bs_ref[...]
    mu2 = jnp.mean(h2, axis=1, keepdims=True)
    d2 = h2 - mu2
    v2 = jnp.mean(d2 * d2, axis=1, keepdims=True)
    si = d2 * lax.rsqrt(v2 + 1e-5) * gs_ref[...] + bes_ref[...]
    a = jnp.maximum(ti + si, 0.0)
    med_ref[...] = jnp.dot(a, W2_ref[...], preferred_element_type=jnp.float32) + b2_ref[...]


def _dense(x2, mm2, Wt, bt, gt, bet, Ws, bs, gs, bes, W2, b2, interpret=False):
    full = lambda i: (0, 0)
    return pl.pallas_call(
        _dense_body,
        grid=(_NT // _T,),
        in_specs=[
            pl.BlockSpec((_T, _D), lambda i: (i, 0)),
            pl.BlockSpec((_T, _D), lambda i: (i, 0)),
            pl.BlockSpec((_D, _D), full),
            pl.BlockSpec((1, _D), full),
            pl.BlockSpec((1, _D), full),
            pl.BlockSpec((1, _D), full),
            pl.BlockSpec((_D, _D), full),
            pl.BlockSpec((1, _D), full),
            pl.BlockSpec((1, _D), full),
            pl.BlockSpec((1, _D), full),
            pl.BlockSpec((_D, _E), full),
            pl.BlockSpec((1, _E), full),
        ],
        out_specs=pl.BlockSpec((_T, _E), lambda i: (i, 0)),
        out_shape=jax.ShapeDtypeStruct((_NT, _E), jnp.float32),
        interpret=interpret,
    )(x2, mm2, Wt, bt, gt, bet, Ws, bs, gs, bes, W2, b2)


@functools.cache
def _build_route_kernel():
    sc_mesh = plsc.VectorSubcoreMesh(
        core_axis_name="c", subcore_axis_name="s",
        num_cores=1, num_subcores=_NSUB)
    return functools.partial(
        pl.kernel,
        out_type=jax.ShapeDtypeStruct((_NT, _E), jnp.float32),
        mesh=sc_mesh,
        compiler_params=pltpu.CompilerParams(needs_layout_passes=False),
        scratch_types=[
        pltpu.VMEM((_TPT, _E), jnp.float32),
        pltpu.VMEM((2, _E), jnp.float32),
        pltpu.VMEM((_NSUB, 2, _E), jnp.float32),
        pltpu.VMEM((_E, _E), jnp.float32),
        pltpu.VMEM((_E, _E), jnp.float32),
        pltpu.VMEM((_E, _E), jnp.float32),
        pltpu.VMEM((_ZR, _E), jnp.float32),
        pltpu.VMEM_SHARED((_NSUB, 2, _E), jnp.float32),
        ],
    )(_route_body)


def _route_body(med_hbm, out_hbm, med_v, selp_v, comb_v, rows_v, g_v,
                ob_v, z_v, shared):
    wid = lax.axis_index("s")
    base = wid * _TPT
    pltpu.sync_copy(med_hbm.at[pl.ds(base, _TPT)], med_v)

    iota = lax.iota(jnp.int32, _E)
    zero = jnp.zeros((_E,), jnp.float32)
    neg = jnp.full((_E,), -3.4e38, jnp.float32)

    def tok(i, carry):
        s1, s2 = carry
        v = med_v[i]
        m1 = jnp.max(v)
        i1 = plsc.all_reduce_ffs(v == m1)
        oh1 = iota == i1
        v2 = jnp.where(oh1, neg, v)
        m2 = jnp.max(v2)
        i2 = plsc.all_reduce_ffs(v2 == m2)
        oh2 = iota == i2
        return jnp.where(oh1, 1.0, s1), jnp.where(oh2, 1.0, s2)

    s1, s2 = lax.fori_loop(0, _TPT, tok, (zero, zero), unroll=4)
    selp_v[0] = s1
    selp_v[1] = s2
    pltpu.sync_copy(selp_v, shared.at[wid])

    def zb(i, _):
        z_v[i] = zero
        return 0
    lax.fori_loop(0, _ZR, zb, 0, unroll=8)
    for k in range(_TPT // _ZR):
        pltpu.sync_copy(z_v, out_hbm.at[pl.ds(base + k * _ZR, _ZR)])

    plsc.subcore_barrier()

    @pl.when(wid == 0)
    def _finish():
        pltpu.sync_copy(shared, comb_v)
        tpb = _NSUB // _B
        den = [jnp.full((_E,), 1e-4, jnp.float32),
               jnp.full((_E,), 1e-4, jnp.float32)]
        ts = []
        for b in range(_B):
            sel = []
            for j in range(_K):
                acc = comb_v[b * tpb, j]
                for t in range(1, tpb):
                    acc = jnp.maximum(acc, comb_v[b * tpb + t, j])
                sel.append(acc)
            pltpu.sync_copy(med_hbm.at[pl.ds(b * _L, _E)], rows_v)
            for l in range(_E):
                v = rows_v[l]
                e = jnp.exp(v - jnp.max(v))
                g_v[l] = e / jnp.sum(e)
            tb = []
            for j in range(_K):
                colj = plsc.load_gather(
                    g_v, [iota, jnp.full((_E,), j, jnp.int32)])
                tj = colj * sel[j]
                den[j] = den[j] + tj
                tb.append(tj)
            ts.append(tb)
        for b in range(_B):
            for l in range(_E):
                ob_v[l] = zero
            for j in range(_K):
                oj = ts[b][j] / den[j] * _CAP
                plsc.store_scatter(
                    ob_v, [iota, jnp.full((_E,), j, jnp.int32)], oj)
            pltpu.sync_copy(ob_v, out_hbm.at[pl.ds(b * _L, _E)])


def kernel(x, Wt, bt, gt, bet, Ws, bs, gs, bes, W2, b2):
    med = _fused_dense(x, Wt, bt.reshape(1, _D), gt.reshape(1, _D),
                       bet.reshape(1, _D), Ws, bs.reshape(1, _D),
                       gs.reshape(1, _D), bes.reshape(1, _D), W2,
                       b2.reshape(1, _E))
    out2 = _build_route_kernel()(med)
    return out2.reshape(_B, _L, _E)


# speedup vs baseline: 1.5869x; 1.0485x over previous
import functools

import jax
import jax.numpy as jnp
from jax import lax
from jax.experimental import pallas as pl
from jax.experimental.pallas import tpu as pltpu
from jax.experimental.pallas import tpu_sc as plsc

_B, _L, _D = 4, 2048, 1024
_E, _K, _KWIN = 16, 2, 25
_PAD = (_KWIN - 1) // 2
_NT = _B * _L
_NSUB = 16
_TPT = _NT // _NSUB
_CAP = 8.0
_ZR = 128


_CH = 512


def _ln(h, g, b):
    mu = jnp.mean(h, axis=1, keepdims=True)
    d = h - mu
    var = jnp.mean(d * d, axis=1, keepdims=True)
    return d * lax.rsqrt(var + 1e-5) * g + b


_MB = 128
_MBW = 160


def _band_mats():
    i = jnp.arange(_MB)[:, None]
    j = jnp.arange(_MBW)[None, :]
    d = j - i
    inv = 1.0 / _KWIN
    Am = jnp.where((d >= 4) & (d <= 28), inv, 0.0).astype(jnp.float32)
    A0 = (jnp.where(j == 0, jnp.maximum(13 - i, 0), 0.0)
          + jnp.where((j >= 1) & (d >= -12) & (d <= 12), 1.0, 0.0)) * inv
    Ab = (jnp.where((d >= 20) & (d <= 44) & (j <= 158), 1.0, 0.0)
          + jnp.where(j == 159, jnp.maximum(i - 114, 0), 0.0)) * inv
    return A0.astype(jnp.float32), Am, Ab.astype(jnp.float32)


def _fused_body(x_ref, Wt_ref, bt_ref, gt_ref, bet_ref, Ws_ref, bs_ref,
                gs_ref, bes_ref, W2_ref, b2_ref, A0_ref, Am_ref, Ab_ref,
                med_ref):
    xb = x_ref[0]
    bt = bt_ref[...]
    gt = gt_ref[...]
    bet = bet_ref[...]
    bs = bs_ref[...]
    gs = gs_ref[...]
    bes = bes_ref[...]
    W2 = W2_ref[...]
    b2 = b2_ref[...]
    A0 = A0_ref[...]
    Am = Am_ref[...]
    Ab = Ab_ref[...]
    nch = _L // _CH
    outs = []
    for c in range(nch):
        s0 = 0 if c == 0 else c * _CH - 16
        s1 = min((c + 1) * _CH + 16, _L)
        seg = xb[s0:s1]
        h1 = jnp.dot(seg, Wt_ref[...], preferred_element_type=jnp.float32)
        h2 = jnp.dot(seg, Ws_ref[...], preferred_element_type=jnp.float32)
        m1b, m2b = [], []
        for s in range(_CH // _MB):
            tok0 = c * _CH + s * _MB
            if c == 0 and s == 0:
                A, b0 = A0, 0
            elif c == nch - 1 and s == _CH // _MB - 1:
                A, b0 = Ab, 368
            else:
                A, b0 = Am, tok0 - 16 - s0
            m1b.append(jnp.dot(A, h1[b0:b0 + _MBW],
                               preferred_element_type=jnp.float32))
            m2b.append(jnp.dot(A, h2[b0:b0 + _MBW],
                               preferred_element_type=jnp.float32))
        m1 = jnp.concatenate(m1b, axis=0)
        m2 = jnp.concatenate(m2b, axis=0)
        t0 = c * _CH - s0
        ti = _ln(h1[t0:t0 + _CH] - m1 + bt, gt, bet)
        si = _ln(m2 + bs, gs, bes)
        a = jnp.maximum(ti + si, 0.0)
        outs.append(jnp.dot(a, W2, preferred_element_type=jnp.float32) + b2)
    med_ref[...] = jnp.concatenate(outs, axis=0)


def _fused_dense(x, Wt, bt, gt, bet, Ws, bs, gs, bes, W2, b2, interpret=False):
    full = lambda b: (0, 0)
    A0, Am, Ab = _band_mats()
    return pl.pallas_call(
        _fused_body,
        grid=(_B,),
        in_specs=[
            pl.BlockSpec((1, _L, _D), lambda b: (b, 0, 0)),
            pl.BlockSpec((_D, _D), full),
            pl.BlockSpec((1, _D), full),
            pl.BlockSpec((1, _D), full),
            pl.BlockSpec((1, _D), full),
            pl.BlockSpec((_D, _D), full),
            pl.BlockSpec((1, _D), full),
            pl.BlockSpec((1, _D), full),
            pl.BlockSpec((1, _D), full),
            pl.BlockSpec((_D, _E), full),
            pl.BlockSpec((1, _E), full),
            pl.BlockSpec((_MB, _MBW), full),
            pl.BlockSpec((_MB, _MBW), full),
            pl.BlockSpec((_MB, _MBW), full),
        ],
        out_specs=pl.BlockSpec((_L, _E), lambda b: (b, 0)),
        out_shape=jax.ShapeDtypeStruct((_NT, _E), jnp.float32),
        interpret=interpret,
    )(x, Wt, bt, gt, bet, Ws, bs, gs, bes, W2, b2, A0, Am, Ab)


def _mavg_body(x_ref, o_ref):
    xb = x_ref[0]
    f = xb.shape[1]
    xp = jnp.concatenate(
        [jnp.broadcast_to(xb[0:1], (_PAD, f)), xb,
         jnp.broadcast_to(xb[_L - 1:_L], (_PAD, f))], axis=0)
    acc = xp[0:_L]
    for d in range(1, _KWIN):
        acc = acc + xp[d:d + _L]
    o_ref[0] = acc * (1.0 / _KWIN)


def _moving_mean(x, interpret=False):
    f = 512
    return pl.pallas_call(
        _mavg_body,
        grid=(_B, _D // f),
        in_specs=[pl.BlockSpec((1, _L, f), lambda b, i: (b, 0, i))],
        out_specs=pl.BlockSpec((1, _L, f), lambda b, i: (b, 0, i)),
        out_shape=jax.ShapeDtypeStruct((_B, _L, _D), jnp.float32),
        interpret=interpret,
    )(x)


_T = 512


def _dense_body(x_ref, mm_ref, Wt_ref, bt_ref, gt_ref, bet_ref,
                Ws_ref, bs_ref, gs_ref, bes_ref, W2_ref, b2_ref, med_ref):
    mmc = mm_ref[...]
    res = x_ref[...] - mmc
    h1 = jnp.dot(res, Wt_ref[...], preferred_element_type=jnp.float32) + bt_ref[...]
    mu1 = jnp.mean(h1, axis=1, keepdims=True)
    d1 = h1 - mu1
    v1 = jnp.mean(d1 * d1, axis=1, keepdims=True)
    ti = d1 * lax.rsqrt(v1 + 1e-5) * gt_ref[...] + bet_ref[...]
    h2 = jnp.dot(mmc, Ws_ref[...], preferred_element_type=jnp.float32) + bs_ref[...]
    mu2 = jnp.mean(h2, axis=1, keepdims=True)
    d2 = h2 - mu2
    v2 = jnp.mean(d2 * d2, axis=1, keepdims=True)
    si = d2 * lax.rsqrt(v2 + 1e-5) * gs_ref[...] + bes_ref[...]
    a = jnp.maximum(ti + si, 0.0)
    med_ref[...] = jnp.dot(a, W2_ref[...], preferred_element_type=jnp.float32) + b2_ref[...]


def _dense(x2, mm2, Wt, bt, gt, bet, Ws, bs, gs, bes, W2, b2, interpret=False):
    full = lambda i: (0, 0)
    return pl.pallas_call(
        _dense_body,
        grid=(_NT // _T,),
        in_specs=[
            pl.BlockSpec((_T, _D), lambda i: (i, 0)),
            pl.BlockSpec((_T, _D), lambda i: (i, 0)),
            pl.BlockSpec((_D, _D), full),
            pl.BlockSpec((1, _D), full),
            pl.BlockSpec((1, _D), full),
            pl.BlockSpec((1, _D), full),
            pl.BlockSpec((_D, _D), full),
            pl.BlockSpec((1, _D), full),
            pl.BlockSpec((1, _D), full),
            pl.BlockSpec((1, _D), full),
            pl.BlockSpec((_D, _E), full),
            pl.BlockSpec((1, _E), full),
        ],
        out_specs=pl.BlockSpec((_T, _E), lambda i: (i, 0)),
        out_shape=jax.ShapeDtypeStruct((_NT, _E), jnp.float32),
        interpret=interpret,
    )(x2, mm2, Wt, bt, gt, bet, Ws, bs, gs, bes, W2, b2)


@functools.cache
def _build_route_kernel():
    sc_mesh = plsc.VectorSubcoreMesh(
        core_axis_name="c", subcore_axis_name="s",
        num_cores=1, num_subcores=_NSUB)
    return functools.partial(
        pl.kernel,
        out_type=jax.ShapeDtypeStruct((_NT, _E), jnp.float32),
        mesh=sc_mesh,
        compiler_params=pltpu.CompilerParams(needs_layout_passes=False),
        scratch_types=[
        pltpu.VMEM((_TPT, _E), jnp.float32),
        pltpu.VMEM((2, _E), jnp.float32),
        pltpu.VMEM((_NSUB, 2, _E), jnp.float32),
        pltpu.VMEM((_E, _E), jnp.float32),
        pltpu.VMEM((_E, _E), jnp.float32),
        pltpu.VMEM((_E, _E), jnp.float32),
        pltpu.VMEM((_ZR, _E), jnp.float32),
        pltpu.VMEM_SHARED((_NSUB, 2, _E), jnp.float32),
        ],
    )(_route_body)


def _route_body(med_hbm, out_hbm, med_v, selp_v, comb_v, rows_v, g_v,
                ob_v, z_v, shared):
    wid = lax.axis_index("s")
    base = wid * _TPT
    pltpu.sync_copy(med_hbm.at[pl.ds(base, _TPT)], med_v)

    iota = lax.iota(jnp.int32, _E)
    zero = jnp.zeros((_E,), jnp.float32)
    neg = jnp.full((_E,), -3.4e38, jnp.float32)

    def tok(i, carry):
        s1, s2 = carry
        v = med_v[i]
        m1 = jnp.max(v)
        i1 = plsc.all_reduce_ffs(v == m1)
        oh1 = iota == i1
        v2 = jnp.where(oh1, neg, v)
        m2 = jnp.max(v2)
        i2 = plsc.all_reduce_ffs(v2 == m2)
        oh2 = iota == i2
        return jnp.where(oh1, 1.0, s1), jnp.where(oh2, 1.0, s2)

    s1, s2 = lax.fori_loop(0, _TPT, tok, (zero, zero), unroll=4)
    selp_v[0] = s1
    selp_v[1] = s2
    pltpu.sync_copy(selp_v, shared.at[wid])

    def zb(i, _):
        z_v[i] = zero
        return 0
    lax.fori_loop(0, _ZR, zb, 0, unroll=8)
    for k in range(_TPT // _ZR):
        pltpu.sync_copy(z_v, out_hbm.at[pl.ds(base + k * _ZR, _ZR)])

    plsc.subcore_barrier()

    @pl.when(wid == 0)
    def _finish():
        pltpu.sync_copy(shared, comb_v)
        tpb = _NSUB // _B
        den = [jnp.full((_E,), 1e-4, jnp.float32),
               jnp.full((_E,), 1e-4, jnp.float32)]
        ts = []
        for b in range(_B):
            sel = []
            for j in range(_K):
                acc = comb_v[b * tpb, j]
                for t in range(1, tpb):
                    acc = jnp.maximum(acc, comb_v[b * tpb + t, j])
                sel.append(acc)
            pltpu.sync_copy(med_hbm.at[pl.ds(b * _L, _E)], rows_v)
            for l in range(_E):
                v = rows_v[l]
                e = jnp.exp(v - jnp.max(v))
                g_v[l] = e / jnp.sum(e)
            tb = []
            for j in range(_K):
                colj = plsc.load_gather(
                    g_v, [iota, jnp.full((_E,), j, jnp.int32)])
                tj = colj * sel[j]
                den[j] = den[j] + tj
                tb.append(tj)
            ts.append(tb)
        for b in range(_B):
            for l in range(_E):
                ob_v[l] = zero
            for j in range(_K):
                oj = ts[b][j] / den[j] * _CAP
                plsc.store_scatter(
                    ob_v, [iota, jnp.full((_E,), j, jnp.int32)], oj)
            pltpu.sync_copy(ob_v, out_hbm.at[pl.ds(b * _L, _E)])


def kernel(x, Wt, bt, gt, bet, Ws, bs, gs, bes, W2, b2):
    med = _fused_dense(x, Wt, bt.reshape(1, _D), gt.reshape(1, _D),
                       bet.reshape(1, _D), Ws, bs.reshape(1, _D),
                       gs.reshape(1, _D), bes.reshape(1, _D), W2,
                       b2.reshape(1, _E))
    out2 = _build_route_kernel()(med)
    return out2.reshape(_B, _L, _E)


# speedup vs baseline: 1.5875x; 1.0004x over previous
import functools

import jax
import jax.numpy as jnp
from jax import lax
from jax.experimental import pallas as pl
from jax.experimental.pallas import tpu as pltpu
from jax.experimental.pallas import tpu_sc as plsc

_B, _L, _D = 4, 2048, 1024
_E, _K, _KWIN = 16, 2, 25
_PAD = (_KWIN - 1) // 2
_NT = _B * _L
_NSUB = 16
_TPT = _NT // _NSUB
_CAP = 8.0
_ZR = 128


_CH = 512


def _ln(h, g, b):
    mu = jnp.mean(h, axis=1, keepdims=True)
    d = h - mu
    var = jnp.mean(d * d, axis=1, keepdims=True)
    return d * lax.rsqrt(var + 1e-5) * g + b


_MB = 128
_MBW = 160


def _band_mats():
    i = jnp.arange(_MB)[:, None]
    j = jnp.arange(_MBW)[None, :]
    d = j - i
    inv = 1.0 / _KWIN
    Am = jnp.where((d >= 4) & (d <= 28), inv, 0.0).astype(jnp.float32)
    A0 = (jnp.where(j == 0, jnp.maximum(13 - i, 0), 0.0)
          + jnp.where((j >= 1) & (d >= -12) & (d <= 12), 1.0, 0.0)) * inv
    Ab = (jnp.where((d >= 20) & (d <= 44) & (j <= 158), 1.0, 0.0)
          + jnp.where(j == 159, jnp.maximum(i - 114, 0), 0.0)) * inv
    return A0.astype(jnp.float32), Am, Ab.astype(jnp.float32)


def _fused_body(x_ref, Wt_ref, bt_ref, gt_ref, bet_ref, Ws_ref, bs_ref,
                gs_ref, bes_ref, W2_ref, b2_ref, A0_ref, Am_ref, Ab_ref,
                med_ref):
    xb = x_ref[0]
    bt = bt_ref[...]
    gt = gt_ref[...]
    bet = bet_ref[...]
    bs = bs_ref[...]
    gs = gs_ref[...]
    bes = bes_ref[...]
    W2 = W2_ref[...]
    b2 = b2_ref[...]
    A0 = A0_ref[...]
    Am = Am_ref[...]
    Ab = Ab_ref[...]
    nch = _L // _CH
    outs = []
    for c in range(nch):
        s0 = 0 if c == 0 else c * _CH - 16
        s1 = min((c + 1) * _CH + 16, _L)
        seg = xb[s0:s1]
        h1 = jnp.dot(seg, Wt_ref[...], preferred_element_type=jnp.float32)
        h2 = jnp.dot(seg, Ws_ref[...], preferred_element_type=jnp.float32)
        m1b, m2b = [], []
        for s in range(_CH // _MB):
            tok0 = c * _CH + s * _MB
            if c == 0 and s == 0:
                A, b0 = A0, 0
            elif c == nch - 1 and s == _CH // _MB - 1:
                A, b0 = Ab, _L - _MBW - s0
            else:
                A, b0 = Am, tok0 - 16 - s0
            m1b.append(jnp.dot(A, h1[b0:b0 + _MBW],
                               preferred_element_type=jnp.float32))
            m2b.append(jnp.dot(A, h2[b0:b0 + _MBW],
                               preferred_element_type=jnp.float32))
        m1 = jnp.concatenate(m1b, axis=0)
        m2 = jnp.concatenate(m2b, axis=0)
        t0 = c * _CH - s0
        ti = _ln(h1[t0:t0 + _CH] - m1 + bt, gt, bet)
        si = _ln(m2 + bs, gs, bes)
        a = jnp.maximum(ti + si, 0.0)
        outs.append(jnp.dot(a, W2, preferred_element_type=jnp.float32) + b2)
    med_ref[...] = jnp.concatenate(outs, axis=0)


def _fused_dense(x, Wt, bt, gt, bet, Ws, bs, gs, bes, W2, b2, interpret=False):
    full = lambda b: (0, 0)
    A0, Am, Ab = _band_mats()
    return pl.pallas_call(
        _fused_body,
        grid=(_B,),
        in_specs=[
            pl.BlockSpec((1, _L, _D), lambda b: (b, 0, 0)),
            pl.BlockSpec((_D, _D), full),
            pl.BlockSpec((1, _D), full),
            pl.BlockSpec((1, _D), full),
            pl.BlockSpec((1, _D), full),
            pl.BlockSpec((_D, _D), full),
            pl.BlockSpec((1, _D), full),
            pl.BlockSpec((1, _D), full),
            pl.BlockSpec((1, _D), full),
            pl.BlockSpec((_D, _E), full),
            pl.BlockSpec((1, _E), full),
            pl.BlockSpec((_MB, _MBW), full),
            pl.BlockSpec((_MB, _MBW), full),
            pl.BlockSpec((_MB, _MBW), full),
        ],
        out_specs=pl.BlockSpec((_L, _E), lambda b: (b, 0)),
        out_shape=jax.ShapeDtypeStruct((_NT, _E), jnp.float32),
        interpret=interpret,
    )(x, Wt, bt, gt, bet, Ws, bs, gs, bes, W2, b2, A0, Am, Ab)


@functools.cache
def _build_route_kernel():
    sc_mesh = plsc.VectorSubcoreMesh(
        core_axis_name="c", subcore_axis_name="s",
        num_cores=1, num_subcores=_NSUB)
    return functools.partial(
        pl.kernel,
        out_type=jax.ShapeDtypeStruct((_NT, _E), jnp.float32),
        mesh=sc_mesh,
        compiler_params=pltpu.CompilerParams(needs_layout_passes=False),
        scratch_types=[
        pltpu.VMEM((_TPT, _E), jnp.float32),
        pltpu.VMEM((2, _E), jnp.float32),
        pltpu.VMEM((_NSUB, 2, _E), jnp.float32),
        pltpu.VMEM((_E, _E), jnp.float32),
        pltpu.VMEM((_E, _E), jnp.float32),
        pltpu.VMEM((_E, _E), jnp.float32),
        pltpu.VMEM((_ZR, _E), jnp.float32),
        pltpu.VMEM_SHARED((_NSUB, 2, _E), jnp.float32),
        ],
    )(_route_body)


def _route_body(med_hbm, out_hbm, med_v, selp_v, comb_v, rows_v, g_v,
                ob_v, z_v, shared):
    wid = lax.axis_index("s")
    base = wid * _TPT
    pltpu.sync_copy(med_hbm.at[pl.ds(base, _TPT)], med_v)

    iota = lax.iota(jnp.int32, _E)
    zero = jnp.zeros((_E,), jnp.float32)
    neg = jnp.full((_E,), -3.4e38, jnp.float32)

    def tok(i, carry):
        s1, s2 = carry
        v = med_v[i]
        m1 = jnp.max(v)
        i1 = plsc.all_reduce_ffs(v == m1)
        oh1 = iota == i1
        v2 = jnp.where(oh1, neg, v)
        m2 = jnp.max(v2)
        i2 = plsc.all_reduce_ffs(v2 == m2)
        oh2 = iota == i2
        return jnp.where(oh1, 1.0, s1), jnp.where(oh2, 1.0, s2)

    s1, s2 = lax.fori_loop(0, _TPT, tok, (zero, zero), unroll=4)
    selp_v[0] = s1
    selp_v[1] = s2
    pltpu.sync_copy(selp_v, shared.at[wid])

    def zb(i, _):
        z_v[i] = zero
        return 0
    lax.fori_loop(0, _ZR, zb, 0, unroll=8)
    for k in range(_TPT // _ZR):
        pltpu.sync_copy(z_v, out_hbm.at[pl.ds(base + k * _ZR, _ZR)])

    plsc.subcore_barrier()

    @pl.when(wid == 0)
    def _finish():
        pltpu.sync_copy(shared, comb_v)
        tpb = _NSUB // _B
        den = [jnp.full((_E,), 1e-4, jnp.float32),
               jnp.full((_E,), 1e-4, jnp.float32)]
        ts = []
        for b in range(_B):
            sel = []
            for j in range(_K):
                acc = comb_v[b * tpb, j]
                for t in range(1, tpb):
                    acc = jnp.maximum(acc, comb_v[b * tpb + t, j])
                sel.append(acc)
            pltpu.sync_copy(med_hbm.at[pl.ds(b * _L, _E)], rows_v)
            for l in range(_E):
                v = rows_v[l]
                e = jnp.exp(v - jnp.max(v))
                g_v[l] = e / jnp.sum(e)
            tb = []
            for j in range(_K):
                colj = plsc.load_gather(
                    g_v, [iota, jnp.full((_E,), j, jnp.int32)])
                tj = colj * sel[j]
                den[j] = den[j] + tj
                tb.append(tj)
            ts.append(tb)
        for b in range(_B):
            for l in range(_E):
                ob_v[l] = zero
            for j in range(_K):
                oj = ts[b][j] / den[j] * _CAP
                plsc.store_scatter(
                    ob_v, [iota, jnp.full((_E,), j, jnp.int32)], oj)
            pltpu.sync_copy(ob_v, out_hbm.at[pl.ds(b * _L, _E)])


def kernel(x, Wt, bt, gt, bet, Ws, bs, gs, bes, W2, b2):
    med = _fused_dense(x, Wt, bt.reshape(1, _D), gt.reshape(1, _D),
                       bet.reshape(1, _D), Ws, bs.reshape(1, _D),
                       gs.reshape(1, _D), bes.reshape(1, _D), W2,
                       b2.reshape(1, _E))
    out2 = _build_route_kernel()(med)
    return out2.reshape(_B, _L, _E)
